# Initial kernel scaffold; baseline (speedup 1.0000x reference)
import jax
import jax.numpy as jnp
from jax.experimental import pallas as pl


def kernel(x, edge_index, edge_weight, W, b, gamma, beta, Wf, bf):
    raise NotImplementedError("write your pallas kernel here")


# speedup vs baseline: 10.2980x; 10.2980x over previous
import functools

import jax
import jax.numpy as jnp
from jax import lax
from jax.experimental import pallas as pl
from jax.experimental.pallas import tpu as pltpu
from jax.experimental.pallas import tpu_sc as plsc

N_NODES = 10000
N_PAD = 10240
SEG = 640
N_EDGES = 320000
CH = 128
K = 80
NC = 2
NS = 16
EPS = 1e-5


def _fast_rsqrt(d):
    i = lax.bitcast_convert_type(d, jnp.int32)
    y = lax.bitcast_convert_type(jnp.int32(0x5F3759DF) - (i >> 1), jnp.float32)
    half = jnp.float32(0.5) * d
    for _ in range(3):
        y = y * (jnp.float32(1.5) - half * y * y)
    return y


def _sc_body(src_hbm, dst_hbm, ew_hbm, x_hbm,
             pout_hbm, dis_hbm,
             idx_s, idx_d, ewv, normv, rows, dis_t, seg_v,
             deg_sh, dis_sh, agg_sh,
             gsem):
    cid = lax.axis_index("c")
    sid = lax.axis_index("s")
    zero16 = jnp.zeros((16,), jnp.float32)

    @pl.loop(0, K * CH // 16)
    def _zrows(i):
        rows[i // (CH // 16), pl.ds((i % (CH // 16)) * 16, 16)] = zero16

    @pl.loop(0, SEG // 16)
    def _zseg(i):
        seg_v[pl.ds(i * 16, 16)] = zero16

    pltpu.sync_copy(seg_v, deg_sh.at[pl.ds(sid * SEG, SEG)])
    for i in range(SEG // K):
        pltpu.sync_copy(rows, agg_sh.at[pl.ds(sid * SEG + i * K, K)])

    plsc.subcore_barrier()

    deg_base = sid * (N_EDGES // NS)

    @pl.loop(0, N_EDGES // NS // K)
    def _deg(ch):
        b = deg_base + ch * K
        pltpu.sync_copy(dst_hbm.at[pl.ds(b, K)], idx_d)
        pltpu.sync_copy(ew_hbm.at[pl.ds(b, K)], ewv)
        pltpu.sync_copy(ewv, deg_sh.at[idx_d], add=True)

    plsc.subcore_barrier()

    pltpu.sync_copy(deg_sh.at[pl.ds(sid * SEG, SEG)], seg_v)

    @pl.loop(0, SEG // 16)
    def _dis(g):
        sl = pl.ds(g * 16, 16)
        d = seg_v[sl] + jnp.float32(1.0)
        seg_v[sl] = _fast_rsqrt(d)

    pltpu.sync_copy(seg_v, dis_sh.at[pl.ds(sid * SEG, SEG)])

    @pl.when(jnp.logical_and(cid == 0, sid < NS - 1))
    def _():
        pltpu.sync_copy(seg_v, dis_hbm.at[pl.ds(sid * SEG, SEG)])

    @pl.when(jnp.logical_and(cid == 0, sid == NS - 1))
    def _():
        tail = N_NODES - (NS - 1) * SEG
        pltpu.sync_copy(seg_v.at[pl.ds(0, tail)],
                        dis_hbm.at[pl.ds(sid * SEG, tail)])

    plsc.subcore_barrier()

    pltpu.sync_copy(dis_sh, dis_t)
    agg_base = cid * (N_EDGES // NC) + sid * (N_EDGES // NC // NS)

    @pl.loop(0, N_EDGES // NC // NS // K)
    def _agg(ch):
        b = agg_base + ch * K
        pltpu.sync_copy(src_hbm.at[pl.ds(b, K)], idx_s)
        pltpu.sync_copy(dst_hbm.at[pl.ds(b, K)], idx_d)
        pltpu.sync_copy(ew_hbm.at[pl.ds(b, K)], ewv)
        gather = pltpu.async_copy(x_hbm.at[idx_s], rows, gsem)
        for g in range(K // 16):
            sl = pl.ds(g * 16, 16)
            s16 = idx_s[sl]
            d16 = idx_d[sl]
            n16 = (plsc.load_gather(dis_t, [s16])
                   * plsc.load_gather(dis_t, [d16]) * ewv[sl])
            normv[sl] = n16
        gather.wait()

        @pl.loop(0, K)
        def _scale(j):
            nj = normv[pl.ds(j, 16)][0]
            for c in range(CH // 16):
                sl = pl.ds(c * 16, 16)
                rows[j, sl] = rows[j, sl] * nj

        pltpu.sync_copy(rows, agg_sh.at[idx_d], add=True)

    plsc.subcore_barrier()

    last = NS - 1
    tail_chunks = (N_NODES - last * SEG) // K

    @pl.when(sid < last)
    def _():
        for i in range(SEG // K):
            r = sid * SEG + i * K
            pltpu.sync_copy(agg_sh.at[pl.ds(r, K)], rows)
            pltpu.sync_copy(rows, pout_hbm.at[cid, pl.ds(r, K)])

    @pl.when(sid == last)
    def _():
        for i in range(tail_chunks):
            r = last * SEG + i * K
            pltpu.sync_copy(agg_sh.at[pl.ds(r, K)], rows)
            pltpu.sync_copy(rows, pout_hbm.at[cid, pl.ds(r, K)])


@functools.partial(jax.jit, static_argnames=())
def _sc_aggregate(src, dst, ew, x):
    mesh = plsc.VectorSubcoreMesh(core_axis_name="c", subcore_axis_name="s")
    f = pl.kernel(
        _sc_body,
        out_type=[
            jax.ShapeDtypeStruct((NC, N_NODES, CH), jnp.float32),
            jax.ShapeDtypeStruct((N_NODES,), jnp.float32),
        ],
        mesh=mesh,
        scratch_types=[
            pltpu.VMEM((K,), jnp.int32),
            pltpu.VMEM((K,), jnp.int32),
            pltpu.VMEM((K,), jnp.float32),
            pltpu.VMEM((K + 16,), jnp.float32),
            pltpu.VMEM((K, CH), jnp.float32),
            pltpu.VMEM((N_PAD,), jnp.float32),
            pltpu.VMEM((SEG,), jnp.float32),
            pltpu.VMEM_SHARED((N_PAD,), jnp.float32),
            pltpu.VMEM_SHARED((N_PAD,), jnp.float32),
            pltpu.VMEM_SHARED((N_PAD, CH), jnp.float32),
            pltpu.SemaphoreType.DMA,
        ],
        compiler_params=pltpu.CompilerParams(needs_layout_passes=False),
    )
    return f(src, dst, ew, x)


def _tc_body(xb, pa, pb, dis, wt, b, gamma, beta, wft, bf, out):
    agg = pa[0] + pb[0]
    sn = dis[...]
    h = agg + xb[...] * (sn * sn)
    h = jnp.dot(h, wt[...], preferred_element_type=jnp.float32) + b[...]
    mu = jnp.mean(h, axis=1, keepdims=True)
    var = jnp.mean((h - mu) * (h - mu), axis=1, keepdims=True)
    h = (h - mu) * lax.rsqrt(var + jnp.float32(EPS)) * gamma[...] + beta[...]
    h = jnp.maximum(h, jnp.float32(0.0))
    out[...] = jnp.dot(h, wft[...], preferred_element_type=jnp.float32) + bf[...]


def _tc_tail(x, parts, dis2d, wt, b2, gamma2, beta2, wft, bf2):
    blk = 1000
    grid = N_NODES // blk
    return pl.pallas_call(
        _tc_body,
        grid=(grid,),
        in_specs=[
            pl.BlockSpec((blk, CH), lambda i: (i, 0)),
            pl.BlockSpec((1, blk, CH), lambda i: (0, i, 0)),
            pl.BlockSpec((1, blk, CH), lambda i: (1, i, 0)),
            pl.BlockSpec((blk, 1), lambda i: (i, 0)),
            pl.BlockSpec((CH, CH), lambda i: (0, 0)),
            pl.BlockSpec((1, CH), lambda i: (0, 0)),
            pl.BlockSpec((1, CH), lambda i: (0, 0)),
            pl.BlockSpec((1, CH), lambda i: (0, 0)),
            pl.BlockSpec((CH, 16), lambda i: (0, 0)),
            pl.BlockSpec((1, 16), lambda i: (0, 0)),
        ],
        out_specs=pl.BlockSpec((blk, 16), lambda i: (i, 0)),
        out_shape=jax.ShapeDtypeStruct((N_NODES, 16), jnp.float32),
    )(x, parts, parts, dis2d, wt, b2, gamma2, beta2, wft, bf2)


def kernel(x, edge_index, edge_weight, W, b, gamma, beta, Wf, bf):
    src = edge_index[0].astype(jnp.int32)
    dst = edge_index[1].astype(jnp.int32)
    parts, dis = _sc_aggregate(src, dst, edge_weight, x)
    return _tc_tail(
        x, parts, dis.reshape(N_NODES, 1), W.T,
        b.reshape(1, CH), gamma.reshape(1, CH), beta.reshape(1, CH),
        Wf.T, bf.reshape(1, 16),
    )


# speedup vs baseline: 30.3616x; 2.9483x over previous
import functools

import jax
import jax.numpy as jnp
from jax import lax
from jax.experimental import pallas as pl
from jax.experimental.pallas import tpu as pltpu
from jax.experimental.pallas import tpu_sc as plsc

N_NODES = 10000
N_PAD = 10240
SEG = 640
N_EDGES = 320000
CH = 128
K = 80
R3 = 3
R4 = 4
NC = 2
NS = 16
EPS = 1e-5


def _fast_rsqrt(d):
    i = lax.bitcast_convert_type(d, jnp.int32)
    y = lax.bitcast_convert_type(jnp.int32(0x5F3759DF) - (i >> 1), jnp.float32)
    half = jnp.float32(0.5) * d
    for _ in range(3):
        y = y * (jnp.float32(1.5) - half * y * y)
    return y


def _sc_body(src_hbm, dst_hbm, ew_hbm, x_hbm,
             pout_hbm, dis_hbm,
             idx_s, idx_d, ewv, normv, rows, dis_t, seg_v,
             deg_sh, dis_sh, agg_sh,
             gsem, ssem, isem):
    cid = lax.axis_index("c")
    sid = lax.axis_index("s")
    zero16 = jnp.zeros((16,), jnp.float32)
    last = NS - 1

    @pl.loop(0, K * CH // 16)
    def _zrows(i):
        rows[0, i // (CH // 16), pl.ds((i % (CH // 16)) * 16, 16)] = zero16

    @pl.loop(0, SEG // 16)
    def _zseg(i):
        seg_v[pl.ds(i * 16, 16)] = zero16

    pltpu.sync_copy(seg_v, deg_sh.at[pl.ds(sid * SEG, SEG)])

    @pl.when(sid < last)
    def _():
        for i in range(SEG // K):
            pltpu.sync_copy(rows.at[0], agg_sh.at[pl.ds(sid * SEG + i * K, K)])

    @pl.when(sid == last)
    def _():
        for i in range((N_NODES - last * SEG) // K):
            pltpu.sync_copy(rows.at[0], agg_sh.at[pl.ds(last * SEG + i * K, K)])

    plsc.subcore_barrier()

    deg_rows = N_EDGES // NS // K
    deg_bch = sid * deg_rows

    def _deg_loads(c, r):
        b = (deg_bch + c) * K
        pltpu.async_copy(dst_hbm.at[pl.ds(b, K)], idx_d.at[r], isem)
        pltpu.async_copy(ew_hbm.at[pl.ds(b, K)], ewv.at[r], isem)

    def _deg_wait_loads(r):
        pltpu.make_async_copy(dst_hbm.at[pl.ds(0, K)], idx_d.at[r], isem).wait()
        pltpu.make_async_copy(ew_hbm.at[pl.ds(0, K)], ewv.at[r], isem).wait()

    def _deg_wait_scatter(r):
        pltpu.make_async_copy(ewv.at[r], deg_sh.at[idx_d.at[r]], ssem).wait()

    def _deg_chunk(c, i, scwait):
        r, r2 = i % R4, (i + 2) % R4
        if scwait:
            _deg_wait_scatter(r2)
        _deg_loads(c + 2, r2)
        _deg_wait_loads(r)
        pltpu.async_copy(ewv.at[r], deg_sh.at[idx_d.at[r]], ssem, add=True)

    _deg_loads(0, 0)
    _deg_loads(1, 1)
    _deg_chunk(0, 0, False)
    _deg_chunk(1, 1, False)

    @pl.loop(0, (deg_rows - 2) // R4)
    def _deg(g):
        for i in range(R4):
            _deg_chunk(2 + g * R4 + i, (2 + i) % R4, True)

    for r in (0, 1):
        _deg_wait_scatter(r)
    for _ in range(2):
        _deg_wait_loads(0)

    plsc.subcore_barrier()

    pltpu.sync_copy(deg_sh.at[pl.ds(sid * SEG, SEG)], seg_v)

    @pl.loop(0, SEG // 16)
    def _dis(g):
        slc = pl.ds(g * 16, 16)
        seg_v[slc] = _fast_rsqrt(seg_v[slc] + jnp.float32(1.0))

    pltpu.sync_copy(seg_v, dis_sh.at[pl.ds(sid * SEG, SEG)])

    @pl.when(jnp.logical_and(cid == 0, sid < last))
    def _():
        pltpu.sync_copy(seg_v, dis_hbm.at[pl.ds(sid * SEG, SEG)])

    @pl.when(jnp.logical_and(cid == 0, sid == last))
    def _():
        tail = N_NODES - last * SEG
        pltpu.sync_copy(seg_v.at[pl.ds(0, tail)],
                        dis_hbm.at[pl.ds(sid * SEG, tail)])

    plsc.subcore_barrier()

    pltpu.sync_copy(dis_sh.at[pl.ds(0, N_NODES)], dis_t)
    agg_rows = N_EDGES // NC // NS // K
    agg_bch = cid * (N_EDGES // NC // K) + sid * agg_rows

    def _agg_loads(c, r):
        b = (agg_bch + c) * K
        pltpu.async_copy(src_hbm.at[pl.ds(b, K)], idx_s.at[r], isem)
        pltpu.async_copy(dst_hbm.at[pl.ds(b, K)], idx_d.at[r], isem)
        pltpu.async_copy(ew_hbm.at[pl.ds(b, K)], ewv.at[r], isem)

    def _agg_wait_loads(r):
        pltpu.make_async_copy(src_hbm.at[pl.ds(0, K)], idx_s.at[r], isem).wait()
        pltpu.make_async_copy(dst_hbm.at[pl.ds(0, K)], idx_d.at[r], isem).wait()
        pltpu.make_async_copy(ew_hbm.at[pl.ds(0, K)], ewv.at[r], isem).wait()

    def _gather(ri, rr):
        pltpu.async_copy(x_hbm.at[idx_s.at[ri]], rows.at[rr], gsem)

    def _wait_gather(rr):
        pltpu.make_async_copy(x_hbm.at[idx_s.at[0]], rows.at[rr], gsem).wait()

    def _wait_scatter(rr):
        pltpu.make_async_copy(rows.at[rr], agg_sh.at[idx_d.at[0]], ssem).wait()

    def _agg_chunk(c, i4, i3, scwait):
        if scwait:
            _wait_scatter((i3 + 1) % R3)
        _agg_loads(c + 2, (i4 + 2) % R4)
        _agg_wait_loads((i4 + 1) % R4)
        _gather((i4 + 1) % R4, (i3 + 1) % R3)
        for g in range(K // 16):
            slc = pl.ds(g * 16, 16)
            n16 = (plsc.load_gather(dis_t, [idx_s[i4, slc]])
                   * plsc.load_gather(dis_t, [idx_d[i4, slc]]) * ewv[i4, slc])
            normv[i4, slc] = n16
        _wait_gather(i3)

        @pl.loop(0, K, unroll=2)
        def _scale(j):
            nj = normv[i4, pl.ds(j, 16)][0]
            for cc in range(CH // 16):
                slc = pl.ds(cc * 16, 16)
                rows[i3, j, slc] = rows[i3, j, slc] * nj

        pltpu.async_copy(rows.at[i3], agg_sh.at[idx_d.at[i4]], ssem, add=True)

    _agg_loads(0, 0)
    _agg_loads(1, 1)
    _agg_wait_loads(0)
    _gather(0, 0)
    for c in range(5):
        _agg_chunk(c, c % R4, c % R3, c >= 2)

    @pl.loop(0, (agg_rows - 5) // 12)
    def _agg(g):
        for i in range(12):
            c = 5 + g * 12 + i
            _agg_chunk(c, (5 + i) % R4, (5 + i) % R3, True)

    for r in (0, 1):
        _wait_scatter((r + 123) % R3)
    _wait_gather(125 % R3)
    _agg_wait_loads(0)

    plsc.subcore_barrier()

    @pl.when(sid < last)
    def _():
        pltpu.sync_copy(agg_sh.at[pl.ds(sid * SEG, SEG)],
                        pout_hbm.at[cid, pl.ds(sid * SEG, SEG)])

    @pl.when(sid == last)
    def _():
        tail = N_NODES - last * SEG
        pltpu.sync_copy(agg_sh.at[pl.ds(last * SEG, tail)],
                        pout_hbm.at[cid, pl.ds(last * SEG, tail)])


def _sc_aggregate(src, dst, ew, x):
    mesh = plsc.VectorSubcoreMesh(core_axis_name="c", subcore_axis_name="s")
    f = pl.kernel(
        _sc_body,
        out_type=[
            jax.ShapeDtypeStruct((NC, N_NODES, CH), jnp.float32),
            jax.ShapeDtypeStruct((N_NODES,), jnp.float32),
        ],
        mesh=mesh,
        scratch_types=[
            pltpu.VMEM((R4, K), jnp.int32),
            pltpu.VMEM((R4, K), jnp.int32),
            pltpu.VMEM((R4, K), jnp.float32),
            pltpu.VMEM((R4, K + 16), jnp.float32),
            pltpu.VMEM((R3, K, CH), jnp.float32),
            pltpu.VMEM((N_NODES,), jnp.float32),
            pltpu.VMEM((SEG,), jnp.float32),
            pltpu.VMEM_SHARED((N_PAD,), jnp.float32),
            pltpu.VMEM_SHARED((N_PAD,), jnp.float32),
            pltpu.VMEM_SHARED((N_NODES, CH), jnp.float32),
            pltpu.SemaphoreType.DMA,
            pltpu.SemaphoreType.DMA,
            pltpu.SemaphoreType.DMA,
        ],
        compiler_params=pltpu.CompilerParams(needs_layout_passes=False),
    )
    return f(src, dst, ew, x)


def _tc_body(xb, pa, pb, dis, wt, b, gamma, beta, wft, bf, out):
    agg = pa[0] + pb[0]
    sn = dis[...]
    h = agg + xb[...] * (sn * sn)
    h = jnp.dot(h, wt[...], preferred_element_type=jnp.float32) + b[...]
    mu = jnp.mean(h, axis=1, keepdims=True)
    var = jnp.mean((h - mu) * (h - mu), axis=1, keepdims=True)
    h = (h - mu) * lax.rsqrt(var + jnp.float32(EPS)) * gamma[...] + beta[...]
    h = jnp.maximum(h, jnp.float32(0.0))
    out[...] = jnp.dot(h, wft[...], preferred_element_type=jnp.float32) + bf[...]


def _tc_tail(x, parts, dis2d, wt, b2, gamma2, beta2, wft, bf2):
    blk = 1000
    grid = N_NODES // blk
    return pl.pallas_call(
        _tc_body,
        grid=(grid,),
        in_specs=[
            pl.BlockSpec((blk, CH), lambda i: (i, 0)),
            pl.BlockSpec((1, blk, CH), lambda i: (0, i, 0)),
            pl.BlockSpec((1, blk, CH), lambda i: (1, i, 0)),
            pl.BlockSpec((blk, 1), lambda i: (i, 0)),
            pl.BlockSpec((CH, CH), lambda i: (0, 0)),
            pl.BlockSpec((1, CH), lambda i: (0, 0)),
            pl.BlockSpec((1, CH), lambda i: (0, 0)),
            pl.BlockSpec((1, CH), lambda i: (0, 0)),
            pl.BlockSpec((CH, 16), lambda i: (0, 0)),
            pl.BlockSpec((1, 16), lambda i: (0, 0)),
        ],
        out_specs=pl.BlockSpec((blk, 16), lambda i: (i, 0)),
        out_shape=jax.ShapeDtypeStruct((N_NODES, 16), jnp.float32),
    )(x, parts, parts, dis2d, wt, b2, gamma2, beta2, wft, bf2)


def kernel(x, edge_index, edge_weight, W, b, gamma, beta, Wf, bf):
    pad = 8 * K
    src = jnp.pad(edge_index[0].astype(jnp.int32), (0, pad))
    dst = jnp.pad(edge_index[1].astype(jnp.int32), (0, pad))
    ew = jnp.pad(edge_weight, (0, pad))
    parts, dis = _sc_aggregate(src, dst, ew, x)
    return _tc_tail(
        x, parts, dis.reshape(N_NODES, 1), W.T,
        b.reshape(1, CH), gamma.reshape(1, CH), beta.reshape(1, CH),
        Wf.T, bf.reshape(1, 16),
    )
